# Initial kernel scaffold; baseline (speedup 1.0000x reference)
import jax
import jax.numpy as jnp
from jax.experimental import pallas as pl


def kernel(x_industry, edge_index_industry, x_pos_corr, edge_index_pos, W1i, a1i_s, a1i_d, b1i, W2i, a2i_s, a2i_d, b2i, W1p, a1p_s, a1p_d, b1p, W2p, a2p_s, a2p_d, b2p, fha, aw, mlp_W, mlp_b):
    raise NotImplementedError("write your pallas kernel here")


# speedup vs baseline: 8.7366x; 8.7366x over previous
import jax
import jax.numpy as jnp
from jax import lax
from jax.experimental import pallas as pl
from jax.experimental.pallas import tpu as pltpu
from jax.experimental.pallas import tpu_sc as plsc

N = 10000
D = 128
HD = 64
NPAD = 10240
E_RAW = 320000
E_FULL = E_RAW + N
EP = 335872
NB_T = 16
NB_C = 2
TCHUNK = EP // NB_T
SB = 512
NBLK_S = TCHUNK // SB
FCHUNK = EP // (NB_C * NB_T)
FB = 128
NBLK_F = FCHUNK // FB
NSLICE = NPAD // NB_T


def _edge_logit(as_v, ad_v, si, di):
    e = plsc.load_gather(as_v, [si]) + plsc.load_gather(ad_v, [di])
    return jnp.maximum(e, e * jnp.float32(0.2))


def _scatter_max(m_ref, idx, val):
    def body(_):
        cur = plsc.load_gather(m_ref, [idx])
        plsc.store_scatter(m_ref, [idx], val, mask=val > cur)
        cur2 = plsc.load_gather(m_ref, [idx])
        return jnp.any(val > cur2)

    lax.while_loop(lambda go: go, body, jnp.bool_(True))


def _gat_sc_body(hL_hbm, hR_hbm, asad_hbm, src_hbm, dst_hbm, out_hbm,
                 as_v, ad_v, m_v, s_v, srcb_v, dstb_v, exb_v,
                 fsrc_v, fdst_v, alpha_v, rows_v, red_v,
                 shm_red, shm_m, shm_s, shm_out, sem):
    cid = lax.axis_index("c")
    sid = lax.axis_index("s")
    NEG = jnp.float32(-1e30)
    nbase = sid * NSLICE

    pltpu.sync_copy(asad_hbm.at[0], as_v)
    pltpu.sync_copy(asad_hbm.at[1], ad_v)

    def fill(i, _):
        m_v[pl.ds(i * 16, 16)] = jnp.full((16,), NEG, jnp.float32)
        s_v[pl.ds(i * 16, 16)] = jnp.zeros((16,), jnp.float32)
        return 0
    lax.fori_loop(0, NPAD // 16, fill, 0)

    def _zero_rows():
        def fillr(r, _):
            z = jnp.zeros((16,), jnp.float32)
            for k in range(HD // 16):
                rows_v[r, pl.ds(k * 16, 16)] = z
            return 0
        lax.fori_loop(0, FB, fillr, 0)

    def _zero_my_out_slice():
        for j in range(NSLICE // FB):
            pltpu.sync_copy(rows_v, shm_out.at[pl.ds(nbase + j * FB, FB)])

    _zero_rows()
    _zero_my_out_slice()
    pltpu.sync_copy(s_v.at[pl.ds(0, NSLICE)], shm_s.at[pl.ds(nbase, NSLICE)])
    plsc.subcore_barrier()

    tbase4 = sid * (TCHUNK // 128)

    def blk1(b, _):
        off4 = tbase4 + b * 4
        pltpu.sync_copy(src_hbm.at[pl.ds(off4, 4)], srcb_v)
        pltpu.sync_copy(dst_hbm.at[pl.ds(off4, 4)], dstb_v)
        for j in range(4):
            def vec1(v, _, j=j):
                si = srcb_v[j, pl.ds(v * 16, 16)]
                di = dstb_v[j, pl.ds(v * 16, 16)]
                _scatter_max(m_v, di, _edge_logit(as_v, ad_v, si, di))
                return 0
            lax.fori_loop(0, 8, vec1, 0)
        return 0
    lax.fori_loop(0, NBLK_S, blk1, 0)

    pltpu.sync_copy(m_v, shm_red.at[sid])
    plsc.subcore_barrier()
    for t in range(NB_T):
        pltpu.sync_copy(shm_red.at[t, pl.ds(nbase, NSLICE)], red_v.at[t])

    def redmax(q, _):
        acc = red_v[0, pl.ds(q * 16, 16)]
        for t in range(1, NB_T):
            acc = jnp.maximum(acc, red_v[t, pl.ds(q * 16, 16)])
        m_v[pl.ds(nbase + q * 16, 16)] = acc
        return 0
    lax.fori_loop(0, NSLICE // 16, redmax, 0)
    pltpu.sync_copy(m_v.at[pl.ds(nbase, NSLICE)], shm_m.at[pl.ds(nbase, NSLICE)])
    plsc.subcore_barrier()
    pltpu.sync_copy(shm_m, m_v)

    def blk2(b, _):
        off4 = tbase4 + b * 4
        pltpu.sync_copy(src_hbm.at[pl.ds(off4, 4)], srcb_v)
        pltpu.sync_copy(dst_hbm.at[pl.ds(off4, 4)], dstb_v)
        for j in range(4):
            def vec2(v, _, j=j):
                si = srcb_v[j, pl.ds(v * 16, 16)]
                di = dstb_v[j, pl.ds(v * 16, 16)]
                e = _edge_logit(as_v, ad_v, si, di)
                mg = plsc.load_gather(m_v, [di])
                exb_v[j, pl.ds(v * 16, 16)] = jnp.exp(e - mg)
                return 0
            lax.fori_loop(0, 8, vec2, 0)
            pltpu.sync_copy(exb_v.at[j], shm_s.at[dstb_v.at[j]], add=True)
        return 0
    lax.fori_loop(0, NBLK_S, blk2, 0)
    plsc.subcore_barrier()
    pltpu.sync_copy(shm_s, s_v)

    wbase = (cid * NB_T + sid) * (FCHUNK // 128)

    for half, h_hbm in ((0, hL_hbm), (1, hR_hbm)):
        def blkf(b, _, half=half, h_hbm=h_hbm):
            pltpu.sync_copy(src_hbm.at[pl.ds(wbase + b, 1)], fsrc_v)
            pltpu.sync_copy(dst_hbm.at[pl.ds(wbase + b, 1)], fdst_v)
            cp = pltpu.async_copy(h_hbm.at[fsrc_v.at[0]], rows_v, sem)

            if half == 0:
                def veca(v, _):
                    si = fsrc_v[0, pl.ds(v * 16, 16)]
                    di = fdst_v[0, pl.ds(v * 16, 16)]
                    e = _edge_logit(as_v, ad_v, si, di)
                    mg = plsc.load_gather(m_v, [di])
                    sg = plsc.load_gather(s_v, [di])
                    alpha_v[pl.ds(b * FB + v * 16, 16)] = (
                        jnp.exp(e - mg) / (sg + jnp.float32(1e-16)))
                    return 0
                lax.fori_loop(0, 8, veca, 0)
            cp.wait()

            def scale(g, _):
                av = alpha_v[pl.ds(b * FB + g * 16, 16)]
                for r2 in range(16):
                    a = av[r2]
                    r = g * 16 + r2
                    for k in range(HD // 16):
                        rows_v[r, pl.ds(k * 16, 16)] = rows_v[r, pl.ds(k * 16, 16)] * a
                return 0
            lax.fori_loop(0, FB // 16, scale, 0)
            pltpu.sync_copy(rows_v, shm_out.at[fdst_v.at[0]], add=True)
            return 0
        lax.fori_loop(0, NBLK_F, blkf, 0)

        plsc.subcore_barrier()
        for j in range(NSLICE // FB):
            pltpu.sync_copy(shm_out.at[pl.ds(nbase + j * FB, FB)], rows_v)
            pltpu.sync_copy(rows_v, out_hbm.at[cid, half, pl.ds(nbase + j * FB, FB)])
        if half == 0:
            _zero_rows()
            _zero_my_out_slice()
            plsc.subcore_barrier()


@jax.jit
def _gat_sc(hL, hR, asad, src2, dst2):
    return pl.kernel(
        _gat_sc_body,
        out_type=jax.ShapeDtypeStruct((2, 2, NPAD, HD), jnp.float32),
        mesh=plsc.VectorSubcoreMesh(core_axis_name="c", subcore_axis_name="s"),
        compiler_params=pltpu.CompilerParams(needs_layout_passes=False,
                                             use_tc_tiling_on_sc=False),
        scratch_types=[
            pltpu.VMEM((NPAD,), jnp.float32),
            pltpu.VMEM((NPAD,), jnp.float32),
            pltpu.VMEM((NPAD,), jnp.float32),
            pltpu.VMEM((NPAD,), jnp.float32),
            pltpu.VMEM((4, 128), jnp.int32),
            pltpu.VMEM((4, 128), jnp.int32),
            pltpu.VMEM((4, 128), jnp.float32),
            pltpu.VMEM((1, 128), jnp.int32),
            pltpu.VMEM((1, 128), jnp.int32),
            pltpu.VMEM((FCHUNK,), jnp.float32),
            pltpu.VMEM((FB, HD), jnp.float32),
            pltpu.VMEM((NB_T, NSLICE), jnp.float32),
            pltpu.VMEM_SHARED((NB_T, NPAD), jnp.float32),
            pltpu.VMEM_SHARED((NPAD,), jnp.float32),
            pltpu.VMEM_SHARED((NPAD,), jnp.float32),
            pltpu.VMEM_SHARED((NPAD, HD), jnp.float32),
            pltpu.SemaphoreType.DMA,
        ],
    )(hL, hR, asad, src2, dst2)


def _transform_body(x_ref, w_ref, aw_ref, hL_ref, hR_ref, asad_ref):
    h = jnp.dot(x_ref[...], w_ref[...], preferred_element_type=jnp.float32)
    hL_ref[...] = h[:, :HD]
    hR_ref[...] = h[:, HD:]
    asad_ref[...] = lax.dot_general(aw_ref[...], h, (((1,), (1,)), ((), ())),
                                    preferred_element_type=jnp.float32)


def _transform(x_pad, W, a_s, a_d):
    aw = jnp.zeros((8, 128), jnp.float32).at[0].set(a_s).at[1].set(a_d)
    R = 2048
    return pl.pallas_call(
        _transform_body,
        grid=(NPAD // R,),
        in_specs=[
            pl.BlockSpec((R, 128), lambda i: (i, 0)),
            pl.BlockSpec((128, 128), lambda i: (0, 0)),
            pl.BlockSpec((8, 128), lambda i: (0, 0)),
        ],
        out_specs=[
            pl.BlockSpec((R, HD), lambda i: (i, 0)),
            pl.BlockSpec((R, HD), lambda i: (i, 0)),
            pl.BlockSpec((8, R), lambda i: (0, i)),
        ],
        out_shape=[
            jax.ShapeDtypeStruct((NPAD, HD), jnp.float32),
            jax.ShapeDtypeStruct((NPAD, HD), jnp.float32),
            jax.ShapeDtypeStruct((8, NPAD), jnp.float32),
        ],
    )(x_pad, W, aw)


def _combine_parts(p_ref):
    return jnp.concatenate([p_ref[0, 0] + p_ref[1, 0],
                            p_ref[0, 1] + p_ref[1, 1]], axis=1)


def _fuse1_body(pi_ref, pp_ref, bi_ref, bp_ref, f0_ref, f1_ref, xf_ref):
    xi = jnp.maximum(_combine_parts(pi_ref) + bi_ref[...], 0.0)
    xp = jnp.maximum(_combine_parts(pp_ref) + bp_ref[...], 0.0)
    ti = jnp.sum(xi * f0_ref[...], axis=1)
    tp = jnp.sum(xp * f1_ref[...], axis=1)
    mx = jnp.maximum(ti, tp)
    e0 = jnp.exp(ti - mx)
    e1 = jnp.exp(tp - mx)
    a0 = e0 / (e0 + e1)
    a1 = e1 / (e0 + e1)
    xf_ref[...] = a0[:, None] * xi + a1[:, None] * xp


def _fuse1(pi, pp, b1i, b1p, fha):
    R = 2048
    return pl.pallas_call(
        _fuse1_body,
        grid=(NPAD // R,),
        in_specs=[
            pl.BlockSpec((2, 2, R, HD), lambda i: (0, 0, i, 0)),
            pl.BlockSpec((2, 2, R, HD), lambda i: (0, 0, i, 0)),
            pl.BlockSpec((128,), lambda i: (0,)),
            pl.BlockSpec((128,), lambda i: (0,)),
            pl.BlockSpec((128,), lambda i: (0,)),
            pl.BlockSpec((128,), lambda i: (0,)),
        ],
        out_specs=pl.BlockSpec((R, 128), lambda i: (i, 0)),
        out_shape=jax.ShapeDtypeStruct((NPAD, 128), jnp.float32),
    )(pi, pp, b1i, b1p, fha[0], fha[1])


def _final_body(pi_ref, pp_ref, bi_ref, bp_ref, w0_ref, w1_ref, mw_ref, y_ref):
    xi = _combine_parts(pi_ref) + bi_ref[...]
    xp = _combine_parts(pp_ref) + bp_ref[...]
    ti = jnp.sum(xi * w0_ref[...], axis=1)
    tp = jnp.sum(xp * w1_ref[...], axis=1)
    mx = jnp.maximum(ti, tp)
    e0 = jnp.exp(ti - mx)
    e1 = jnp.exp(tp - mx)
    a0 = e0 / (e0 + e1)
    a1 = e1 / (e0 + e1)
    x = a0[:, None] * xi + a1[:, None] * xp
    y_ref[...] = jnp.dot(x, mw_ref[...], preferred_element_type=jnp.float32)


def _final(pi2, pp2, b2i, b2p, aw, mlp_W):
    mw = jnp.zeros((128, 128), jnp.float32).at[:, 0].set(mlp_W[:, 0])
    R = 2048
    return pl.pallas_call(
        _final_body,
        grid=(NPAD // R,),
        in_specs=[
            pl.BlockSpec((2, 2, R, HD), lambda i: (0, 0, i, 0)),
            pl.BlockSpec((2, 2, R, HD), lambda i: (0, 0, i, 0)),
            pl.BlockSpec((128,), lambda i: (0,)),
            pl.BlockSpec((128,), lambda i: (0,)),
            pl.BlockSpec((128,), lambda i: (0,)),
            pl.BlockSpec((128,), lambda i: (0,)),
            pl.BlockSpec((128, 128), lambda i: (0, 0)),
        ],
        out_specs=pl.BlockSpec((R, 128), lambda i: (i, 0)),
        out_shape=jax.ShapeDtypeStruct((NPAD, 128), jnp.float32),
    )(pi2, pp2, b2i, b2p, aw[0], aw[1], mw)


def _prep_edges(edge_index):
    loop = jnp.arange(N, dtype=jnp.int32)
    padv = jnp.full((EP - E_FULL,), N, jnp.int32)
    src = jnp.concatenate([edge_index[0].astype(jnp.int32), loop, padv])
    dst = jnp.concatenate([edge_index[1].astype(jnp.int32), loop, padv])
    return src.reshape(EP // 128, 128), dst.reshape(EP // 128, 128)


def kernel(x_industry, edge_index_industry, x_pos_corr, edge_index_pos,
           W1i, a1i_s, a1i_d, b1i, W2i, a2i_s, a2i_d, b2i,
           W1p, a1p_s, a1p_d, b1p, W2p, a2p_s, a2p_d, b2p,
           fha, aw, mlp_W, mlp_b):
    xi_pad = jnp.pad(x_industry, ((0, NPAD - N), (0, 0)))
    xp_pad = jnp.pad(x_pos_corr, ((0, NPAD - N), (0, 0)))
    src_i, dst_i = _prep_edges(edge_index_industry)
    src_p, dst_p = _prep_edges(edge_index_pos)

    h1iL, h1iR, asad1i = _transform(xi_pad, W1i, a1i_s, a1i_d)
    pi = _gat_sc(h1iL, h1iR, asad1i, src_i, dst_i)
    h1pL, h1pR, asad1p = _transform(xp_pad, W1p, a1p_s, a1p_d)
    pp = _gat_sc(h1pL, h1pR, asad1p, src_p, dst_p)
    xf = _fuse1(pi, pp, b1i, b1p, fha)

    h2iL, h2iR, asad2i = _transform(xf, W2i, a2i_s, a2i_d)
    pi2 = _gat_sc(h2iL, h2iR, asad2i, src_i, dst_i)
    h2pL, h2pR, asad2p = _transform(xf, W2p, a2p_s, a2p_d)
    pp2 = _gat_sc(h2pL, h2pR, asad2p, src_p, dst_p)

    y = _final(pi2, pp2, b2i, b2p, aw, mlp_W)
    return y[:N, :1] + mlp_b


# speedup vs baseline: 15.4241x; 1.7655x over previous
import jax
import jax.numpy as jnp
from jax import lax
from jax.experimental import pallas as pl
from jax.experimental.pallas import tpu as pltpu
from jax.experimental.pallas import tpu_sc as plsc

N = 10000
D = 128
HD = 32
NPAD = 10240
E_RAW = 320000
E_FULL = E_RAW + N
EP = 335872
NB_T = 16
NB_C = 2
TCHUNK = EP // NB_T
SB = 512
NBLK_S = TCHUNK // SB
FCHUNK = EP // (NB_C * NB_T)
FB = 128
NBLK_F = FCHUNK // FB
NSLICE = NPAD // NB_T


def _edge_logit(as_v, ad_v, si, di):
    e = plsc.load_gather(as_v, [si]) + plsc.load_gather(ad_v, [di])
    return jnp.maximum(e, e * jnp.float32(0.2))


def _scatter_max(m_ref, idx, val):
    def body(_):
        cur = plsc.load_gather(m_ref, [idx])
        plsc.store_scatter(m_ref, [idx], val, mask=val > cur)
        cur2 = plsc.load_gather(m_ref, [idx])
        return jnp.any(val > cur2)

    lax.while_loop(lambda go: go, body, jnp.bool_(True))


def _gat_sc_body(h0_hbm, h1_hbm, h2_hbm, h3_hbm, asad_hbm, src_hbm, dst_hbm, out_hbm,
                 as_v, ad_v, m_v, s_v, srcb_v, dstb_v, exb_v,
                 fsrc_v, fdst_v, alpha_v, rows_v, red_v,
                 shm_red, shm_m, shm_s, shm_out, sem, sem2):
    cid = lax.axis_index("c")
    sid = lax.axis_index("s")
    NEG = jnp.float32(-1e30)
    nbase = sid * NSLICE

    pltpu.sync_copy(asad_hbm.at[0], as_v)
    pltpu.sync_copy(asad_hbm.at[1], ad_v)

    def fill(i, _):
        m_v[pl.ds(i * 16, 16)] = jnp.full((16,), NEG, jnp.float32)
        s_v[pl.ds(i * 16, 16)] = jnp.zeros((16,), jnp.float32)
        return 0
    lax.fori_loop(0, NPAD // 16, fill, 0)

    def _zero_rows():
        def fillr(r, _):
            z = jnp.zeros((16,), jnp.float32)
            for k in range(HD // 16):
                rows_v[0, r, pl.ds(k * 16, 16)] = z
            return 0
        lax.fori_loop(0, FB, fillr, 0)

    def _zero_my_out_slice():
        for j in range(NSLICE // FB):
            pltpu.sync_copy(rows_v.at[0], shm_out.at[pl.ds(nbase + j * FB, FB)])

    _zero_rows()
    _zero_my_out_slice()
    pltpu.sync_copy(s_v.at[pl.ds(0, NSLICE)], shm_s.at[pl.ds(nbase, NSLICE)])
    plsc.subcore_barrier()

    tbase4 = sid * (TCHUNK // 128)

    def blk1(b, _):
        off4 = tbase4 + b * 4
        pltpu.sync_copy(src_hbm.at[pl.ds(off4, 4)], srcb_v)
        pltpu.sync_copy(dst_hbm.at[pl.ds(off4, 4)], dstb_v)
        for j in range(4):
            def vec1(v, _, j=j):
                si = srcb_v[j, pl.ds(v * 16, 16)]
                di = dstb_v[j, pl.ds(v * 16, 16)]
                _scatter_max(m_v, di, _edge_logit(as_v, ad_v, si, di))
                return 0
            lax.fori_loop(0, 8, vec1, 0)
        return 0
    lax.fori_loop(0, NBLK_S, blk1, 0)

    pltpu.sync_copy(m_v, shm_red.at[sid])
    plsc.subcore_barrier()
    for t in range(NB_T):
        pltpu.sync_copy(shm_red.at[t, pl.ds(nbase, NSLICE)], red_v.at[t])

    def redmax(q, _):
        acc = red_v[0, pl.ds(q * 16, 16)]
        for t in range(1, NB_T):
            acc = jnp.maximum(acc, red_v[t, pl.ds(q * 16, 16)])
        m_v[pl.ds(nbase + q * 16, 16)] = acc
        return 0
    lax.fori_loop(0, NSLICE // 16, redmax, 0)
    pltpu.sync_copy(m_v.at[pl.ds(nbase, NSLICE)], shm_m.at[pl.ds(nbase, NSLICE)])
    plsc.subcore_barrier()
    pltpu.sync_copy(shm_m, m_v)

    def blk2(b, _):
        off4 = tbase4 + b * 4
        pltpu.sync_copy(src_hbm.at[pl.ds(off4, 4)], srcb_v)
        pltpu.sync_copy(dst_hbm.at[pl.ds(off4, 4)], dstb_v)
        for j in range(4):
            def vec2(v, _, j=j):
                si = srcb_v[j, pl.ds(v * 16, 16)]
                di = dstb_v[j, pl.ds(v * 16, 16)]
                e = _edge_logit(as_v, ad_v, si, di)
                mg = plsc.load_gather(m_v, [di])
                exb_v[j, pl.ds(v * 16, 16)] = jnp.exp(e - mg)
                return 0
            lax.fori_loop(0, 8, vec2, 0)
            pltpu.sync_copy(exb_v.at[j], shm_s.at[dstb_v.at[j]], add=True)
        return 0
    lax.fori_loop(0, NBLK_S, blk2, 0)
    plsc.subcore_barrier()
    pltpu.sync_copy(shm_s, s_v)

    wbase = (cid * NB_T + sid) * (FCHUNK // 128)
    pltpu.sync_copy(src_hbm.at[pl.ds(wbase, NBLK_F)], fsrc_v)
    pltpu.sync_copy(dst_hbm.at[pl.ds(wbase, NBLK_F)], fdst_v)

    def alf(b, _):
        def veca(v, _):
            si = fsrc_v[b, pl.ds(v * 16, 16)]
            di = fdst_v[b, pl.ds(v * 16, 16)]
            e = _edge_logit(as_v, ad_v, si, di)
            mg = plsc.load_gather(m_v, [di])
            sg = plsc.load_gather(s_v, [di])
            alpha_v[pl.ds(b * FB + v * 16, 16)] = (
                jnp.exp(e - mg) / (sg + jnp.float32(1e-16)))
            return 0
        lax.fori_loop(0, 8, veca, 0)
        return 0
    lax.fori_loop(0, NBLK_F, alf, 0)

    def _scale_bank(bank, boff):
        def scale(g, _):
            av = alpha_v[pl.ds(boff + g * 16, 16)]
            for r2 in range(16):
                a = av[r2]
                r = g * 16 + r2
                for k in range(HD // 16):
                    rows_v[bank, r, pl.ds(k * 16, 16)] = (
                        rows_v[bank, r, pl.ds(k * 16, 16)] * a)
            return 0
        lax.fori_loop(0, FB // 16, scale, 0)

    sems = (sem, sem2)

    for half, h_hbm in enumerate((h0_hbm, h1_hbm, h2_hbm, h3_hbm)):
        def _issue(b, bank, h_hbm=h_hbm):
            return pltpu.async_copy(h_hbm.at[fsrc_v.at[b]], rows_v.at[bank],
                                    sems[bank])

        def _wait(bank, h_hbm=h_hbm):
            pltpu.make_async_copy(h_hbm.at[fsrc_v.at[0]], rows_v.at[bank],
                                  sems[bank]).wait()

        _issue(0, 0)

        def gloop(g, _):
            b = g * 2
            _issue(b + 1, 1)
            _wait(0)
            _scale_bank(0, b * FB)
            pltpu.sync_copy(rows_v.at[0], shm_out.at[fdst_v.at[b]], add=True)

            @pl.when(g < NBLK_F // 2 - 1)
            def _():
                _issue(b + 2, 0)
            _wait(1)
            _scale_bank(1, (b + 1) * FB)
            pltpu.sync_copy(rows_v.at[1], shm_out.at[fdst_v.at[b + 1]], add=True)
            return 0
        lax.fori_loop(0, NBLK_F // 2, gloop, 0)

        plsc.subcore_barrier()
        for j in range(NSLICE // FB):
            pltpu.sync_copy(shm_out.at[pl.ds(nbase + j * FB, FB)], rows_v.at[0])
            pltpu.sync_copy(rows_v.at[0], out_hbm.at[cid, half, pl.ds(nbase + j * FB, FB)])
        if half < 3:
            _zero_rows()
            _zero_my_out_slice()
            plsc.subcore_barrier()


@jax.jit
def _gat_sc(h0, h1, h2, h3, asad, src2, dst2):
    return pl.kernel(
        _gat_sc_body,
        out_type=jax.ShapeDtypeStruct((2, 4, NPAD, HD), jnp.float32),
        mesh=plsc.VectorSubcoreMesh(core_axis_name="c", subcore_axis_name="s"),
        compiler_params=pltpu.CompilerParams(needs_layout_passes=False,
                                             use_tc_tiling_on_sc=False),
        scratch_types=[
            pltpu.VMEM((NPAD,), jnp.float32),
            pltpu.VMEM((NPAD,), jnp.float32),
            pltpu.VMEM((NPAD,), jnp.float32),
            pltpu.VMEM((NPAD,), jnp.float32),
            pltpu.VMEM((4, 128), jnp.int32),
            pltpu.VMEM((4, 128), jnp.int32),
            pltpu.VMEM((4, 128), jnp.float32),
            pltpu.VMEM((NBLK_F, 128), jnp.int32),
            pltpu.VMEM((NBLK_F, 128), jnp.int32),
            pltpu.VMEM((FCHUNK,), jnp.float32),
            pltpu.VMEM((2, FB, HD), jnp.float32),
            pltpu.VMEM((NB_T, NSLICE), jnp.float32),
            pltpu.VMEM_SHARED((NB_T, NPAD), jnp.float32),
            pltpu.VMEM_SHARED((NPAD,), jnp.float32),
            pltpu.VMEM_SHARED((NPAD,), jnp.float32),
            pltpu.VMEM_SHARED((NPAD, HD), jnp.float32),
            pltpu.SemaphoreType.DMA,
            pltpu.SemaphoreType.DMA,
        ],
    )(h0, h1, h2, h3, asad, src2, dst2)


def _transform_body(x_ref, w_ref, aw_ref, h0_ref, h1_ref, h2_ref, h3_ref, asad_ref):
    h = jnp.dot(x_ref[...], w_ref[...], preferred_element_type=jnp.float32)
    h0_ref[...] = h[:, 0 * HD:1 * HD]
    h1_ref[...] = h[:, 1 * HD:2 * HD]
    h2_ref[...] = h[:, 2 * HD:3 * HD]
    h3_ref[...] = h[:, 3 * HD:4 * HD]
    asad_ref[...] = lax.dot_general(aw_ref[...], h, (((1,), (1,)), ((), ())),
                                    preferred_element_type=jnp.float32)


def _transform(x_pad, W, a_s, a_d):
    aw = jnp.zeros((8, 128), jnp.float32).at[0].set(a_s).at[1].set(a_d)
    R = 2048
    return pl.pallas_call(
        _transform_body,
        grid=(NPAD // R,),
        in_specs=[
            pl.BlockSpec((R, 128), lambda i: (i, 0)),
            pl.BlockSpec((128, 128), lambda i: (0, 0)),
            pl.BlockSpec((8, 128), lambda i: (0, 0)),
        ],
        out_specs=[
            pl.BlockSpec((R, HD), lambda i: (i, 0)),
            pl.BlockSpec((R, HD), lambda i: (i, 0)),
            pl.BlockSpec((R, HD), lambda i: (i, 0)),
            pl.BlockSpec((R, HD), lambda i: (i, 0)),
            pl.BlockSpec((8, R), lambda i: (0, i)),
        ],
        out_shape=[
            jax.ShapeDtypeStruct((NPAD, HD), jnp.float32),
            jax.ShapeDtypeStruct((NPAD, HD), jnp.float32),
            jax.ShapeDtypeStruct((NPAD, HD), jnp.float32),
            jax.ShapeDtypeStruct((NPAD, HD), jnp.float32),
            jax.ShapeDtypeStruct((8, NPAD), jnp.float32),
        ],
    )(x_pad, W, aw)


def _combine_parts(p_ref):
    return jnp.concatenate([p_ref[0, q] + p_ref[1, q] for q in range(4)],
                           axis=1)


def _fuse1_body(pi_ref, pp_ref, bi_ref, bp_ref, f0_ref, f1_ref, xf_ref):
    xi = jnp.maximum(_combine_parts(pi_ref) + bi_ref[...], 0.0)
    xp = jnp.maximum(_combine_parts(pp_ref) + bp_ref[...], 0.0)
    ti = jnp.sum(xi * f0_ref[...], axis=1)
    tp = jnp.sum(xp * f1_ref[...], axis=1)
    mx = jnp.maximum(ti, tp)
    e0 = jnp.exp(ti - mx)
    e1 = jnp.exp(tp - mx)
    a0 = e0 / (e0 + e1)
    a1 = e1 / (e0 + e1)
    xf_ref[...] = a0[:, None] * xi + a1[:, None] * xp


def _fuse1(pi, pp, b1i, b1p, fha):
    R = 2048
    return pl.pallas_call(
        _fuse1_body,
        grid=(NPAD // R,),
        in_specs=[
            pl.BlockSpec((2, 4, R, HD), lambda i: (0, 0, i, 0)),
            pl.BlockSpec((2, 4, R, HD), lambda i: (0, 0, i, 0)),
            pl.BlockSpec((128,), lambda i: (0,)),
            pl.BlockSpec((128,), lambda i: (0,)),
            pl.BlockSpec((128,), lambda i: (0,)),
            pl.BlockSpec((128,), lambda i: (0,)),
        ],
        out_specs=pl.BlockSpec((R, 128), lambda i: (i, 0)),
        out_shape=jax.ShapeDtypeStruct((NPAD, 128), jnp.float32),
    )(pi, pp, b1i, b1p, fha[0], fha[1])


def _final_body(pi_ref, pp_ref, bi_ref, bp_ref, w0_ref, w1_ref, mw_ref, y_ref):
    xi = _combine_parts(pi_ref) + bi_ref[...]
    xp = _combine_parts(pp_ref) + bp_ref[...]
    ti = jnp.sum(xi * w0_ref[...], axis=1)
    tp = jnp.sum(xp * w1_ref[...], axis=1)
    mx = jnp.maximum(ti, tp)
    e0 = jnp.exp(ti - mx)
    e1 = jnp.exp(tp - mx)
    a0 = e0 / (e0 + e1)
    a1 = e1 / (e0 + e1)
    x = a0[:, None] * xi + a1[:, None] * xp
    y_ref[...] = jnp.dot(x, mw_ref[...], preferred_element_type=jnp.float32)


def _final(pi2, pp2, b2i, b2p, aw, mlp_W):
    mw = jnp.zeros((128, 128), jnp.float32).at[:, 0].set(mlp_W[:, 0])
    R = 2048
    return pl.pallas_call(
        _final_body,
        grid=(NPAD // R,),
        in_specs=[
            pl.BlockSpec((2, 4, R, HD), lambda i: (0, 0, i, 0)),
            pl.BlockSpec((2, 4, R, HD), lambda i: (0, 0, i, 0)),
            pl.BlockSpec((128,), lambda i: (0,)),
            pl.BlockSpec((128,), lambda i: (0,)),
            pl.BlockSpec((128,), lambda i: (0,)),
            pl.BlockSpec((128,), lambda i: (0,)),
            pl.BlockSpec((128, 128), lambda i: (0, 0)),
        ],
        out_specs=pl.BlockSpec((R, 128), lambda i: (i, 0)),
        out_shape=jax.ShapeDtypeStruct((NPAD, 128), jnp.float32),
    )(pi2, pp2, b2i, b2p, aw[0], aw[1], mw)


def _prep_edges(edge_index):
    loop = jnp.arange(N, dtype=jnp.int32)
    padv = jnp.full((EP - E_FULL,), N, jnp.int32)
    src = jnp.concatenate([edge_index[0].astype(jnp.int32), loop, padv])
    dst = jnp.concatenate([edge_index[1].astype(jnp.int32), loop, padv])
    return src.reshape(EP // 128, 128), dst.reshape(EP // 128, 128)


def kernel(x_industry, edge_index_industry, x_pos_corr, edge_index_pos,
           W1i, a1i_s, a1i_d, b1i, W2i, a2i_s, a2i_d, b2i,
           W1p, a1p_s, a1p_d, b1p, W2p, a2p_s, a2p_d, b2p,
           fha, aw, mlp_W, mlp_b):
    xi_pad = jnp.pad(x_industry, ((0, NPAD - N), (0, 0)))
    xp_pad = jnp.pad(x_pos_corr, ((0, NPAD - N), (0, 0)))
    src_i, dst_i = _prep_edges(edge_index_industry)
    src_p, dst_p = _prep_edges(edge_index_pos)

    *h1i, asad1i = _transform(xi_pad, W1i, a1i_s, a1i_d)
    pi = _gat_sc(*h1i, asad1i, src_i, dst_i)
    *h1p, asad1p = _transform(xp_pad, W1p, a1p_s, a1p_d)
    pp = _gat_sc(*h1p, asad1p, src_p, dst_p)
    xf = _fuse1(pi, pp, b1i, b1p, fha)

    *h2i, asad2i = _transform(xf, W2i, a2i_s, a2i_d)
    pi2 = _gat_sc(*h2i, asad2i, src_i, dst_i)
    *h2p, asad2p = _transform(xf, W2p, a2p_s, a2p_d)
    pp2 = _gat_sc(*h2p, asad2p, src_p, dst_p)

    y = _final(pi2, pp2, b2i, b2p, aw, mlp_W)
    return y[:N, :1] + mlp_b


# speedup vs baseline: 15.7132x; 1.0187x over previous
import jax
import jax.numpy as jnp
from jax import lax
from jax.experimental import pallas as pl
from jax.experimental.pallas import tpu as pltpu
from jax.experimental.pallas import tpu_sc as plsc

N = 10000
D = 128
HD = 32
NPAD = 10240
E_RAW = 320000
E_FULL = E_RAW + N
EP = 335872
NB_T = 16
NB_C = 2
TCHUNK = EP // NB_T
SB = 512
NBLK_S = TCHUNK // SB
FCHUNK = EP // (NB_C * NB_T)
FB = 128
NBLK_F = FCHUNK // FB
NSLICE = NPAD // NB_T


def _edge_logit(as_v, ad_v, si, di):
    e = plsc.load_gather(as_v, [si]) + plsc.load_gather(ad_v, [di])
    return jnp.maximum(e, e * jnp.float32(0.2))


def _scatter_max(m_ref, idx, val):
    def body(_):
        cur = plsc.load_gather(m_ref, [idx])
        plsc.store_scatter(m_ref, [idx], val, mask=val > cur)
        cur2 = plsc.load_gather(m_ref, [idx])
        return jnp.any(val > cur2)

    lax.while_loop(lambda go: go, body, jnp.bool_(True))


def _gat_sc_body(h0_hbm, h1_hbm, h2_hbm, h3_hbm, asad_hbm, src_hbm, dst_hbm, out_hbm,
                 as_v, ad_v, m_v, s_v,
                 fsrc_v, fdst_v, alpha_v, rows_v, red_v,
                 shm_red, shm_m, shm_out, sem, sem2):
    cid = lax.axis_index("c")
    sid = lax.axis_index("s")
    NEG = jnp.float32(-1e30)
    nbase = sid * NSLICE

    pltpu.sync_copy(asad_hbm.at[0], as_v)
    pltpu.sync_copy(asad_hbm.at[1], ad_v)

    def fill(i, _):
        m_v[pl.ds(i * 16, 16)] = jnp.full((16,), NEG, jnp.float32)
        s_v[pl.ds(i * 16, 16)] = jnp.zeros((16,), jnp.float32)
        return 0
    lax.fori_loop(0, NPAD // 16, fill, 0)

    def _zero_rows():
        def fillr(r, _):
            z = jnp.zeros((16,), jnp.float32)
            for k in range(HD // 16):
                rows_v[0, r, pl.ds(k * 16, 16)] = z
            return 0
        lax.fori_loop(0, FB, fillr, 0)

    def _zero_my_out_slice():
        for j in range(NSLICE // FB):
            pltpu.sync_copy(rows_v.at[0], shm_out.at[pl.ds(nbase + j * FB, FB)])

    _zero_rows()
    _zero_my_out_slice()

    tbase = sid * (TCHUNK // 128)

    def _edge_scan(vec_body):
        for sb in range(TCHUNK // 128 // NBLK_F):
            pltpu.sync_copy(src_hbm.at[pl.ds(tbase + sb * NBLK_F, NBLK_F)], fsrc_v)
            pltpu.sync_copy(dst_hbm.at[pl.ds(tbase + sb * NBLK_F, NBLK_F)], fdst_v)

            def blk(b, _):
                def vec(v, _):
                    si = fsrc_v[b, pl.ds(v * 16, 16)]
                    di = fdst_v[b, pl.ds(v * 16, 16)]
                    vec_body(si, di)
                    return 0
                lax.fori_loop(0, 8, vec, 0)
                return 0
            lax.fori_loop(0, NBLK_F, blk, 0)

    def _tree_reduce(x_v, combine):
        pltpu.sync_copy(x_v, shm_red.at[sid])
        plsc.subcore_barrier()
        for t in range(NB_T):
            pltpu.sync_copy(shm_red.at[t, pl.ds(nbase, NSLICE)], red_v.at[t])

        def red(q, _):
            acc = red_v[0, pl.ds(q * 16, 16)]
            for t in range(1, NB_T):
                acc = combine(acc, red_v[t, pl.ds(q * 16, 16)])
            x_v[pl.ds(nbase + q * 16, 16)] = acc
            return 0
        lax.fori_loop(0, NSLICE // 16, red, 0)
        pltpu.sync_copy(x_v.at[pl.ds(nbase, NSLICE)], shm_m.at[pl.ds(nbase, NSLICE)])
        plsc.subcore_barrier()
        pltpu.sync_copy(shm_m, x_v)

    def vec1(si, di):
        _scatter_max(m_v, di, _edge_logit(as_v, ad_v, si, di))
    _edge_scan(vec1)
    _tree_reduce(m_v, jnp.maximum)

    plsc.subcore_barrier()

    def vec2(si, di):
        e = _edge_logit(as_v, ad_v, si, di)
        mg = plsc.load_gather(m_v, [di])
        plsc.addupdate_scatter(s_v, [di], jnp.exp(e - mg))
    _edge_scan(vec2)
    _tree_reduce(s_v, lambda a, b: a + b)

    wbase = (cid * NB_T + sid) * (FCHUNK // 128)
    pltpu.sync_copy(src_hbm.at[pl.ds(wbase, NBLK_F)], fsrc_v)
    pltpu.sync_copy(dst_hbm.at[pl.ds(wbase, NBLK_F)], fdst_v)

    def alf(b, _):
        def veca(v, _):
            si = fsrc_v[b, pl.ds(v * 16, 16)]
            di = fdst_v[b, pl.ds(v * 16, 16)]
            e = _edge_logit(as_v, ad_v, si, di)
            mg = plsc.load_gather(m_v, [di])
            sg = plsc.load_gather(s_v, [di])
            alpha_v[pl.ds(b * FB + v * 16, 16)] = (
                jnp.exp(e - mg) / (sg + jnp.float32(1e-16)))
            return 0
        lax.fori_loop(0, 8, veca, 0)
        return 0
    lax.fori_loop(0, NBLK_F, alf, 0)

    def _scale_bank(bank, boff):
        def scale(r, _):
            a = plsc.load_gather(alpha_v, [jnp.full((16,), boff + r, jnp.int32)])
            for k in range(HD // 16):
                rows_v[bank, r, pl.ds(k * 16, 16)] = (
                    rows_v[bank, r, pl.ds(k * 16, 16)] * a)
            return 0
        lax.fori_loop(0, FB, scale, 0)

    sems = (sem, sem2)

    for half, h_hbm in enumerate((h0_hbm, h1_hbm, h2_hbm, h3_hbm)):
        def _issue(b, bank, h_hbm=h_hbm):
            return pltpu.async_copy(h_hbm.at[fsrc_v.at[b]], rows_v.at[bank],
                                    sems[bank])

        def _wait(bank, h_hbm=h_hbm):
            pltpu.make_async_copy(h_hbm.at[fsrc_v.at[0]], rows_v.at[bank],
                                  sems[bank]).wait()

        _issue(0, 0)

        def gloop(g, _):
            b = g * 2
            _issue(b + 1, 1)
            _wait(0)
            _scale_bank(0, b * FB)
            pltpu.sync_copy(rows_v.at[0], shm_out.at[fdst_v.at[b]], add=True)

            @pl.when(g < NBLK_F // 2 - 1)
            def _():
                _issue(b + 2, 0)
            _wait(1)
            _scale_bank(1, (b + 1) * FB)
            pltpu.sync_copy(rows_v.at[1], shm_out.at[fdst_v.at[b + 1]], add=True)
            return 0
        lax.fori_loop(0, NBLK_F // 2, gloop, 0)

        plsc.subcore_barrier()
        for j in range(NSLICE // FB):
            pltpu.sync_copy(shm_out.at[pl.ds(nbase + j * FB, FB)], rows_v.at[0])
            pltpu.sync_copy(rows_v.at[0], out_hbm.at[cid, half, pl.ds(nbase + j * FB, FB)])
        if half < 3:
            _zero_rows()
            _zero_my_out_slice()
            plsc.subcore_barrier()


@jax.jit
def _gat_sc(h0, h1, h2, h3, asad, src2, dst2):
    return pl.kernel(
        _gat_sc_body,
        out_type=jax.ShapeDtypeStruct((2, 4, NPAD, HD), jnp.float32),
        mesh=plsc.VectorSubcoreMesh(core_axis_name="c", subcore_axis_name="s"),
        compiler_params=pltpu.CompilerParams(needs_layout_passes=False,
                                             use_tc_tiling_on_sc=False),
        scratch_types=[
            pltpu.VMEM((NPAD,), jnp.float32),
            pltpu.VMEM((NPAD,), jnp.float32),
            pltpu.VMEM((NPAD,), jnp.float32),
            pltpu.VMEM((NPAD,), jnp.float32),
            pltpu.VMEM((NBLK_F, 128), jnp.int32),
            pltpu.VMEM((NBLK_F, 128), jnp.int32),
            pltpu.VMEM((FCHUNK,), jnp.float32),
            pltpu.VMEM((2, FB, HD), jnp.float32),
            pltpu.VMEM((NB_T, NSLICE), jnp.float32),
            pltpu.VMEM_SHARED((NB_T, NPAD), jnp.float32),
            pltpu.VMEM_SHARED((NPAD,), jnp.float32),
            pltpu.VMEM_SHARED((NPAD, HD), jnp.float32),
            pltpu.SemaphoreType.DMA,
            pltpu.SemaphoreType.DMA,
        ],
    )(h0, h1, h2, h3, asad, src2, dst2)


def _transform_body(x_ref, w_ref, aw_ref, h0_ref, h1_ref, h2_ref, h3_ref, asad_ref):
    h = jnp.dot(x_ref[...], w_ref[...], preferred_element_type=jnp.float32)
    h0_ref[...] = h[:, 0 * HD:1 * HD]
    h1_ref[...] = h[:, 1 * HD:2 * HD]
    h2_ref[...] = h[:, 2 * HD:3 * HD]
    h3_ref[...] = h[:, 3 * HD:4 * HD]
    asad_ref[...] = lax.dot_general(aw_ref[...], h, (((1,), (1,)), ((), ())),
                                    preferred_element_type=jnp.float32)


def _transform(x_pad, W, a_s, a_d):
    aw = jnp.zeros((8, 128), jnp.float32).at[0].set(a_s).at[1].set(a_d)
    R = 2048
    return pl.pallas_call(
        _transform_body,
        grid=(NPAD // R,),
        in_specs=[
            pl.BlockSpec((R, 128), lambda i: (i, 0)),
            pl.BlockSpec((128, 128), lambda i: (0, 0)),
            pl.BlockSpec((8, 128), lambda i: (0, 0)),
        ],
        out_specs=[
            pl.BlockSpec((R, HD), lambda i: (i, 0)),
            pl.BlockSpec((R, HD), lambda i: (i, 0)),
            pl.BlockSpec((R, HD), lambda i: (i, 0)),
            pl.BlockSpec((R, HD), lambda i: (i, 0)),
            pl.BlockSpec((8, R), lambda i: (0, i)),
        ],
        out_shape=[
            jax.ShapeDtypeStruct((NPAD, HD), jnp.float32),
            jax.ShapeDtypeStruct((NPAD, HD), jnp.float32),
            jax.ShapeDtypeStruct((NPAD, HD), jnp.float32),
            jax.ShapeDtypeStruct((NPAD, HD), jnp.float32),
            jax.ShapeDtypeStruct((8, NPAD), jnp.float32),
        ],
    )(x_pad, W, aw)


def _combine_parts(p_ref):
    return jnp.concatenate([p_ref[0, q] + p_ref[1, q] for q in range(4)],
                           axis=1)


def _fuse1_body(pi_ref, pp_ref, bi_ref, bp_ref, f0_ref, f1_ref, xf_ref):
    xi = jnp.maximum(_combine_parts(pi_ref) + bi_ref[...], 0.0)
    xp = jnp.maximum(_combine_parts(pp_ref) + bp_ref[...], 0.0)
    ti = jnp.sum(xi * f0_ref[...], axis=1)
    tp = jnp.sum(xp * f1_ref[...], axis=1)
    mx = jnp.maximum(ti, tp)
    e0 = jnp.exp(ti - mx)
    e1 = jnp.exp(tp - mx)
    a0 = e0 / (e0 + e1)
    a1 = e1 / (e0 + e1)
    xf_ref[...] = a0[:, None] * xi + a1[:, None] * xp


def _fuse1(pi, pp, b1i, b1p, fha):
    R = 2048
    return pl.pallas_call(
        _fuse1_body,
        grid=(NPAD // R,),
        in_specs=[
            pl.BlockSpec((2, 4, R, HD), lambda i: (0, 0, i, 0)),
            pl.BlockSpec((2, 4, R, HD), lambda i: (0, 0, i, 0)),
            pl.BlockSpec((128,), lambda i: (0,)),
            pl.BlockSpec((128,), lambda i: (0,)),
            pl.BlockSpec((128,), lambda i: (0,)),
            pl.BlockSpec((128,), lambda i: (0,)),
        ],
        out_specs=pl.BlockSpec((R, 128), lambda i: (i, 0)),
        out_shape=jax.ShapeDtypeStruct((NPAD, 128), jnp.float32),
    )(pi, pp, b1i, b1p, fha[0], fha[1])


def _final_body(pi_ref, pp_ref, bi_ref, bp_ref, w0_ref, w1_ref, mw_ref, y_ref):
    xi = _combine_parts(pi_ref) + bi_ref[...]
    xp = _combine_parts(pp_ref) + bp_ref[...]
    ti = jnp.sum(xi * w0_ref[...], axis=1)
    tp = jnp.sum(xp * w1_ref[...], axis=1)
    mx = jnp.maximum(ti, tp)
    e0 = jnp.exp(ti - mx)
    e1 = jnp.exp(tp - mx)
    a0 = e0 / (e0 + e1)
    a1 = e1 / (e0 + e1)
    x = a0[:, None] * xi + a1[:, None] * xp
    y_ref[...] = jnp.dot(x, mw_ref[...], preferred_element_type=jnp.float32)


def _final(pi2, pp2, b2i, b2p, aw, mlp_W):
    mw = jnp.zeros((128, 128), jnp.float32).at[:, 0].set(mlp_W[:, 0])
    R = 2048
    return pl.pallas_call(
        _final_body,
        grid=(NPAD // R,),
        in_specs=[
            pl.BlockSpec((2, 4, R, HD), lambda i: (0, 0, i, 0)),
            pl.BlockSpec((2, 4, R, HD), lambda i: (0, 0, i, 0)),
            pl.BlockSpec((128,), lambda i: (0,)),
            pl.BlockSpec((128,), lambda i: (0,)),
            pl.BlockSpec((128,), lambda i: (0,)),
            pl.BlockSpec((128,), lambda i: (0,)),
            pl.BlockSpec((128, 128), lambda i: (0, 0)),
        ],
        out_specs=pl.BlockSpec((R, 128), lambda i: (i, 0)),
        out_shape=jax.ShapeDtypeStruct((NPAD, 128), jnp.float32),
    )(pi2, pp2, b2i, b2p, aw[0], aw[1], mw)


def _prep_edges(edge_index):
    loop = jnp.arange(N, dtype=jnp.int32)
    padv = jnp.full((EP - E_FULL,), N, jnp.int32)
    src = jnp.concatenate([edge_index[0].astype(jnp.int32), loop, padv])
    dst = jnp.concatenate([edge_index[1].astype(jnp.int32), loop, padv])
    return src.reshape(EP // 128, 128), dst.reshape(EP // 128, 128)


def kernel(x_industry, edge_index_industry, x_pos_corr, edge_index_pos,
           W1i, a1i_s, a1i_d, b1i, W2i, a2i_s, a2i_d, b2i,
           W1p, a1p_s, a1p_d, b1p, W2p, a2p_s, a2p_d, b2p,
           fha, aw, mlp_W, mlp_b):
    xi_pad = jnp.pad(x_industry, ((0, NPAD - N), (0, 0)))
    xp_pad = jnp.pad(x_pos_corr, ((0, NPAD - N), (0, 0)))
    src_i, dst_i = _prep_edges(edge_index_industry)
    src_p, dst_p = _prep_edges(edge_index_pos)

    *h1i, asad1i = _transform(xi_pad, W1i, a1i_s, a1i_d)
    pi = _gat_sc(*h1i, asad1i, src_i, dst_i)
    *h1p, asad1p = _transform(xp_pad, W1p, a1p_s, a1p_d)
    pp = _gat_sc(*h1p, asad1p, src_p, dst_p)
    xf = _fuse1(pi, pp, b1i, b1p, fha)

    *h2i, asad2i = _transform(xf, W2i, a2i_s, a2i_d)
    pi2 = _gat_sc(*h2i, asad2i, src_i, dst_i)
    *h2p, asad2p = _transform(xf, W2p, a2p_s, a2p_d)
    pp2 = _gat_sc(*h2p, asad2p, src_p, dst_p)

    y = _final(pi2, pp2, b2i, b2p, aw, mlp_W)
    return y[:N, :1] + mlp_b
